# initial kernel scaffold (unmeasured)
import jax
import jax.numpy as jnp
from jax import lax
from jax.experimental import pallas as pl
from jax.experimental.pallas import tpu as pltpu

N_Y = 2
T_LOC = 512
D = 1024
E_LOC = 4
E = 8
F = 2048


def _peer():
    my_x = lax.axis_index("x")
    my_y = lax.axis_index("y")
    my_z = lax.axis_index("z")
    return my_y, (my_x, 1 - my_y, my_z)


def _exchange(x, router):

    def body(x_ref, r_ref, xa_ref, rf_ref, send_sems, recv_sems):
        my_y, peer = _peer()

        barrier = pltpu.get_barrier_semaphore()
        pl.semaphore_signal(
            barrier, inc=1, device_id=peer, device_id_type=pl.DeviceIdType.MESH
        )
        pl.semaphore_wait(barrier, 1)

        xa_ref[my_y] = x_ref[...]
        rf_ref[my_y] = r_ref[...]

        rdma_x = pltpu.make_async_remote_copy(
            src_ref=x_ref,
            dst_ref=xa_ref.at[my_y],
            send_sem=send_sems.at[0],
            recv_sem=recv_sems.at[0],
            device_id=peer,
            device_id_type=pl.DeviceIdType.MESH,
        )
        rdma_r = pltpu.make_async_remote_copy(
            src_ref=r_ref,
            dst_ref=rf_ref.at[my_y],
            send_sem=send_sems.at[1],
            recv_sem=recv_sems.at[1],
            device_id=peer,
            device_id_type=pl.DeviceIdType.MESH,
        )
        rdma_x.start()
        rdma_r.start()
        rdma_x.wait()
        rdma_r.wait()

    return pl.pallas_call(
        body,
        out_shape=(
            jax.ShapeDtypeStruct((N_Y, T_LOC, D), jnp.float32),
            jax.ShapeDtypeStruct((N_Y, D, E_LOC), jnp.float32),
        ),
        in_specs=[
            pl.BlockSpec(memory_space=pltpu.VMEM),
            pl.BlockSpec(memory_space=pltpu.VMEM),
        ],
        out_specs=(
            pl.BlockSpec(memory_space=pltpu.VMEM),
            pl.BlockSpec(memory_space=pltpu.VMEM),
        ),
        scratch_shapes=[
            pltpu.SemaphoreType.DMA((2,)),
            pltpu.SemaphoreType.DMA((2,)),
        ],
        compiler_params=pltpu.CompilerParams(collective_id=0),
    )(x, router)


def _compute(x_all, r_all, W1, W2):

    def body(xa_ref, rf_ref, w1_ref, w2_ref, out_ref):
        e = pl.program_id(0)
        my_y = lax.axis_index("y")
        ge = my_y * E_LOC + e

        r_full = jnp.concatenate([rf_ref[0], rf_ref[1]], axis=1)

        for b in range(N_Y):
            xb = xa_ref[b]
            g = jnp.dot(
                xb,
                r_full,
                preferred_element_type=jnp.float32,
                precision=lax.Precision.HIGHEST,
            )
            idx = lax.broadcasted_iota(jnp.int32, (T_LOC, E), 1)
            v1 = jnp.max(g, axis=1, keepdims=True)
            t1 = jnp.min(jnp.where(g == v1, idx, E), axis=1, keepdims=True)
            gm = jnp.where(idx == t1, -1e30, g)
            v2 = jnp.max(gm, axis=1, keepdims=True)
            t2 = jnp.min(jnp.where(gm == v2, idx, E), axis=1, keepdims=True)
            r = jnp.exp(v2 - v1)
            wt1 = 1.0 / (1.0 + r)
            wt2 = r / (1.0 + r)
            we = jnp.where(t1 == ge, wt1, 0.0) + jnp.where(t2 == ge, wt2, 0.0)

            h = jnp.maximum(
                jnp.dot(xb, w1_ref[0], preferred_element_type=jnp.float32), 0.0
            )
            yb = jnp.dot(h, w2_ref[0], preferred_element_type=jnp.float32)
            contrib = yb * we

            @pl.when(e == 0)
            def _():
                out_ref[b] = contrib

            @pl.when(e > 0)
            def _():
                out_ref[b] = out_ref[b] + contrib

    return pl.pallas_call(
        body,
        grid=(E_LOC,),
        out_shape=jax.ShapeDtypeStruct((N_Y, T_LOC, D), jnp.float32),
        in_specs=[
            pl.BlockSpec((N_Y, T_LOC, D), lambda e: (0, 0, 0)),
            pl.BlockSpec((N_Y, D, E_LOC), lambda e: (0, 0, 0)),
            pl.BlockSpec((1, D, F), lambda e: (e, 0, 0)),
            pl.BlockSpec((1, F, D), lambda e: (e, 0, 0)),
        ],
        out_specs=pl.BlockSpec((N_Y, T_LOC, D), lambda e: (0, 0, 0)),
        compiler_params=pltpu.CompilerParams(
            dimension_semantics=("arbitrary",)
        ),
    )(x_all, r_all, W1, W2)


def _combine(partial):

    def body(p_ref, out_ref, recv_buf, send_sem, recv_sem):
        my_y, peer = _peer()

        barrier = pltpu.get_barrier_semaphore()
        pl.semaphore_signal(
            barrier, inc=1, device_id=peer, device_id_type=pl.DeviceIdType.MESH
        )
        pl.semaphore_wait(barrier, 1)

        rdma = pltpu.make_async_remote_copy(
            src_ref=p_ref.at[1 - my_y],
            dst_ref=recv_buf,
            send_sem=send_sem,
            recv_sem=recv_sem,
            device_id=peer,
            device_id_type=pl.DeviceIdType.MESH,
        )
        rdma.start()
        rdma.wait()

        out_ref[...] = p_ref[my_y] + recv_buf[...]

    return pl.pallas_call(
        body,
        out_shape=jax.ShapeDtypeStruct((T_LOC, D), jnp.float32),
        in_specs=[pl.BlockSpec(memory_space=pltpu.VMEM)],
        out_specs=pl.BlockSpec(memory_space=pltpu.VMEM),
        scratch_shapes=[
            pltpu.VMEM((T_LOC, D), jnp.float32),
            pltpu.SemaphoreType.DMA,
            pltpu.SemaphoreType.DMA,
        ],
        compiler_params=pltpu.CompilerParams(collective_id=1),
    )(partial)


def kernel(x, router, W1, W2):
    x_all, r_all = _exchange(x, router)
    partial = _compute(x_all, r_all, W1, W2)
    return _combine(partial)


# baseline (device time: 131676 ns/iter reference)
import jax
import jax.numpy as jnp
from jax import lax
from jax.experimental import pallas as pl
from jax.experimental.pallas import tpu as pltpu

N_Y = 2
T_LOC = 512
D = 1024
E_LOC = 4
E = 8
F = 2048


def _peer():
    my_x = lax.axis_index("x")
    my_y = lax.axis_index("y")
    my_z = lax.axis_index("z")
    return my_y, (my_x, 1 - my_y, my_z)


def _exchange(x, router):

    def body(x_ref, r_ref, xa_ref, rf_ref, send_sems, recv_sems):
        my_y, peer = _peer()

        barrier = pltpu.get_barrier_semaphore()
        pl.semaphore_signal(
            barrier, inc=1, device_id=peer, device_id_type=pl.DeviceIdType.MESH
        )
        pl.semaphore_wait(barrier, 1)

        xa_ref[my_y] = x_ref[...]
        rf_ref[my_y] = r_ref[...]

        rdma_x = pltpu.make_async_remote_copy(
            src_ref=x_ref,
            dst_ref=xa_ref.at[my_y],
            send_sem=send_sems.at[0],
            recv_sem=recv_sems.at[0],
            device_id=peer,
            device_id_type=pl.DeviceIdType.MESH,
        )
        rdma_r = pltpu.make_async_remote_copy(
            src_ref=r_ref,
            dst_ref=rf_ref.at[my_y],
            send_sem=send_sems.at[1],
            recv_sem=recv_sems.at[1],
            device_id=peer,
            device_id_type=pl.DeviceIdType.MESH,
        )
        rdma_x.start()
        rdma_r.start()
        rdma_x.wait()
        rdma_r.wait()

    return pl.pallas_call(
        body,
        out_shape=(
            jax.ShapeDtypeStruct((N_Y, T_LOC, D), jnp.float32),
            jax.ShapeDtypeStruct((N_Y, D, E_LOC), jnp.float32),
        ),
        in_specs=[
            pl.BlockSpec(memory_space=pltpu.VMEM),
            pl.BlockSpec(memory_space=pltpu.VMEM),
        ],
        out_specs=(
            pl.BlockSpec(memory_space=pltpu.VMEM),
            pl.BlockSpec(memory_space=pltpu.VMEM),
        ),
        scratch_shapes=[
            pltpu.SemaphoreType.DMA((2,)),
            pltpu.SemaphoreType.DMA((2,)),
        ],
        compiler_params=pltpu.CompilerParams(collective_id=0),
    )(x, router)


def _compute(x_all, r_all, W1, W2):

    def body(xa_ref, rf_ref, w1_ref, w2_ref, out_ref):
        e = pl.program_id(0)
        my_y = lax.axis_index("y")
        ge = my_y * E_LOC + e

        r_full = jnp.concatenate([rf_ref[0], rf_ref[1]], axis=1)

        for b in range(N_Y):
            xb = xa_ref[b]
            g = jnp.dot(
                xb,
                r_full,
                preferred_element_type=jnp.float32,
                precision=lax.Precision.HIGHEST,
            )
            idx = lax.broadcasted_iota(jnp.int32, (T_LOC, E), 1)
            v1 = jnp.max(g, axis=1, keepdims=True)
            t1 = jnp.min(jnp.where(g == v1, idx, E), axis=1, keepdims=True)
            gm = jnp.where(idx == t1, -1e30, g)
            v2 = jnp.max(gm, axis=1, keepdims=True)
            t2 = jnp.min(jnp.where(gm == v2, idx, E), axis=1, keepdims=True)
            r = jnp.exp(v2 - v1)
            wt1 = 1.0 / (1.0 + r)
            wt2 = r / (1.0 + r)
            we = jnp.where(t1 == ge, wt1, 0.0) + jnp.where(t2 == ge, wt2, 0.0)

            h = jnp.maximum(
                jnp.dot(xb, w1_ref[0], preferred_element_type=jnp.float32), 0.0
            )
            yb = jnp.dot(h, w2_ref[0], preferred_element_type=jnp.float32)
            contrib = yb * we

            @pl.when(e == 0)
            def _():
                out_ref[b] = contrib

            @pl.when(e > 0)
            def _():
                out_ref[b] = out_ref[b] + contrib

    return pl.pallas_call(
        body,
        grid=(E_LOC,),
        out_shape=jax.ShapeDtypeStruct((N_Y, T_LOC, D), jnp.float32),
        in_specs=[
            pl.BlockSpec((N_Y, T_LOC, D), lambda e: (0, 0, 0)),
            pl.BlockSpec((N_Y, D, E_LOC), lambda e: (0, 0, 0)),
            pl.BlockSpec((1, D, F), lambda e: (e, 0, 0)),
            pl.BlockSpec((1, F, D), lambda e: (e, 0, 0)),
        ],
        out_specs=pl.BlockSpec((N_Y, T_LOC, D), lambda e: (0, 0, 0)),
        compiler_params=pltpu.CompilerParams(
            dimension_semantics=("arbitrary",),
            vmem_limit_bytes=100 * 1024 * 1024,
        ),
    )(x_all, r_all, W1, W2)


def _combine(partial):

    def body(p_ref, out_ref, recv_buf, send_sem, recv_sem):
        my_y, peer = _peer()

        barrier = pltpu.get_barrier_semaphore()
        pl.semaphore_signal(
            barrier, inc=1, device_id=peer, device_id_type=pl.DeviceIdType.MESH
        )
        pl.semaphore_wait(barrier, 1)

        rdma = pltpu.make_async_remote_copy(
            src_ref=p_ref.at[1 - my_y],
            dst_ref=recv_buf,
            send_sem=send_sem,
            recv_sem=recv_sem,
            device_id=peer,
            device_id_type=pl.DeviceIdType.MESH,
        )
        rdma.start()
        rdma.wait()

        out_ref[...] = p_ref[my_y] + recv_buf[...]

    return pl.pallas_call(
        body,
        out_shape=jax.ShapeDtypeStruct((T_LOC, D), jnp.float32),
        in_specs=[pl.BlockSpec(memory_space=pltpu.VMEM)],
        out_specs=pl.BlockSpec(memory_space=pltpu.VMEM),
        scratch_shapes=[
            pltpu.VMEM((T_LOC, D), jnp.float32),
            pltpu.SemaphoreType.DMA,
            pltpu.SemaphoreType.DMA,
        ],
        compiler_params=pltpu.CompilerParams(collective_id=1),
    )(partial)


def kernel(x, router, W1, W2):
    x_all, r_all = _exchange(x, router)
    partial = _compute(x_all, r_all, W1, W2)
    return _combine(partial)


# device time: 93320 ns/iter; 1.4110x vs baseline; 1.4110x over previous
import jax
import jax.numpy as jnp
from jax import lax
from jax.experimental import pallas as pl
from jax.experimental.pallas import tpu as pltpu

N_Y = 2
T_LOC = 512
D = 1024
E_LOC = 4
E = 8
F = 2048

CAP = 192


def _peer():
    my_x = lax.axis_index("x")
    my_y = lax.axis_index("y")
    my_z = lax.axis_index("z")
    return my_y, (my_x, 1 - my_y, my_z)


def _route(xb, r_full):
    g = jnp.dot(
        xb,
        r_full,
        preferred_element_type=jnp.float32,
        precision=lax.Precision.HIGHEST,
    )
    idx = lax.broadcasted_iota(jnp.int32, g.shape, 1)
    v1 = jnp.max(g, axis=1, keepdims=True)
    t1 = jnp.min(jnp.where(g == v1, idx, E), axis=1, keepdims=True)
    gm = jnp.where(idx == t1, -1e30, g)
    v2 = jnp.max(gm, axis=1, keepdims=True)
    t2 = jnp.min(jnp.where(gm == v2, idx, E), axis=1, keepdims=True)
    r = jnp.exp(v2 - v1)
    wt1 = 1.0 / (1.0 + r)
    wt2 = r / (1.0 + r)
    return t1, t2, wt1, wt2


def kernel(x, router, W1, W2):
    def body(
        x_ref,
        r_ref,
        w1_hbm,
        w2_hbm,
        out_ref,
        w1b,
        w2b,
        xa_ref,
        rf_ref,
        pp_ref,
        recv_ref,
        sr_sems,
        sx_sems,
        sp_sems,
        w_sems,
    ):
        my_y, peer = _peer()

        def load(e, slot):
            c1 = pltpu.make_async_copy(w1_hbm.at[e], w1b.at[slot], w_sems.at[2 * slot])
            c2 = pltpu.make_async_copy(w2_hbm.at[e], w2b.at[slot], w_sems.at[2 * slot + 1])
            c1.start()
            c2.start()
            return (c1, c2)

        def load_wait(cpair):
            cpair[0].wait()
            cpair[1].wait()

        row_i = lax.broadcasted_iota(jnp.int32, (T_LOC, T_LOC), 0)
        col_i = lax.broadcasted_iota(jnp.int32, (T_LOC, T_LOC), 1)
        tri = jnp.where(row_i >= col_i, 1.0, 0.0).astype(jnp.float32)
        slot_i = lax.broadcasted_iota(jnp.int32, (T_LOC, CAP), 1)

        def expert_contrib(xb, slot, ge, t1, t2, wt1, wt2):
            we = jnp.where(t1 == ge, wt1, 0.0) + jnp.where(t2 == ge, wt2, 0.0)
            mask = jnp.logical_or(t1 == ge, t2 == ge)
            maskf = jnp.where(mask, 1.0, 0.0).astype(jnp.float32)
            pos = jnp.dot(tri, maskf, preferred_element_type=jnp.float32)
            pos_i = pos.astype(jnp.int32)
            pt = jnp.where(
                jnp.logical_and(slot_i == pos_i - 1, mask), 1.0, 0.0
            ).astype(jnp.float32)
            xg = lax.dot_general(
                pt,
                xb,
                (((0,), (0,)), ((), ())),
                preferred_element_type=jnp.float32,
            )
            h = jnp.maximum(
                jnp.dot(xg, w1b[slot], preferred_element_type=jnp.float32), 0.0
            )
            y = jnp.dot(h, w2b[slot], preferred_element_type=jnp.float32)
            return jnp.dot(pt * we, y, preferred_element_type=jnp.float32)

        barrier = pltpu.get_barrier_semaphore()
        pl.semaphore_signal(
            barrier, inc=1, device_id=peer, device_id_type=pl.DeviceIdType.MESH
        )
        pl.semaphore_wait(barrier, 1)

        rf_ref[my_y] = r_ref[...]

        rdma_r = pltpu.make_async_remote_copy(
            src_ref=r_ref,
            dst_ref=rf_ref.at[my_y],
            send_sem=sr_sems.at[0],
            recv_sem=sr_sems.at[1],
            device_id=peer,
            device_id_type=pl.DeviceIdType.MESH,
        )
        rdma_x = pltpu.make_async_remote_copy(
            src_ref=x_ref,
            dst_ref=xa_ref,
            send_sem=sx_sems.at[0],
            recv_sem=sx_sems.at[1],
            device_id=peer,
            device_id_type=pl.DeviceIdType.MESH,
        )
        rdma_r.start()
        rdma_x.start()
        rdma_r.wait()
        r_full = jnp.concatenate([rf_ref[0], rf_ref[1]], axis=1)

        l0 = load(0, 0)
        l1 = load(1, 1)
        xb = x_ref[...]
        t1, t2, wt1, wt2 = _route(xb, r_full)

        rdma_x.wait()
        xp = xa_ref[...]
        t1p, t2p, wt1p, wt2p = _route(xp, r_full)
        load_wait(l0)
        acc_peer = expert_contrib(xp, 0, my_y * E_LOC + 0, t1p, t2p, wt1p, wt2p)
        load_wait(l1)
        acc_peer += expert_contrib(xp, 1, my_y * E_LOC + 1, t1p, t2p, wt1p, wt2p)
        l2 = load(2, 0)
        load_wait(l2)
        acc_peer += expert_contrib(xp, 0, my_y * E_LOC + 2, t1p, t2p, wt1p, wt2p)
        l3 = load(3, 1)
        load_wait(l3)
        acc_peer += expert_contrib(xp, 1, my_y * E_LOC + 3, t1p, t2p, wt1p, wt2p)

        pp_ref[...] = acc_peer
        rdma_p = pltpu.make_async_remote_copy(
            src_ref=pp_ref,
            dst_ref=recv_ref,
            send_sem=sp_sems.at[0],
            recv_sem=sp_sems.at[1],
            device_id=peer,
            device_id_type=pl.DeviceIdType.MESH,
        )
        rdma_p.start()

        acc_mine = expert_contrib(xb, 0, my_y * E_LOC + 2, t1, t2, wt1, wt2)
        acc_mine += expert_contrib(xb, 1, my_y * E_LOC + 3, t1, t2, wt1, wt2)
        l4 = load(0, 0)
        load_wait(l4)
        acc_mine += expert_contrib(xb, 0, my_y * E_LOC + 0, t1, t2, wt1, wt2)
        l5 = load(1, 1)
        load_wait(l5)
        acc_mine += expert_contrib(xb, 1, my_y * E_LOC + 1, t1, t2, wt1, wt2)

        rdma_p.wait()
        out_ref[...] = acc_mine + recv_ref[...]

    return pl.pallas_call(
        body,
        out_shape=jax.ShapeDtypeStruct((T_LOC, D), jnp.float32),
        in_specs=[
            pl.BlockSpec(memory_space=pltpu.VMEM),
            pl.BlockSpec(memory_space=pltpu.VMEM),
            pl.BlockSpec(memory_space=pl.ANY),
            pl.BlockSpec(memory_space=pl.ANY),
        ],
        out_specs=pl.BlockSpec(memory_space=pltpu.VMEM),
        scratch_shapes=[
            pltpu.VMEM((2, D, F), jnp.float32),
            pltpu.VMEM((2, F, D), jnp.float32),
            pltpu.VMEM((T_LOC, D), jnp.float32),
            pltpu.VMEM((N_Y, D, E_LOC), jnp.float32),
            pltpu.VMEM((T_LOC, D), jnp.float32),
            pltpu.VMEM((T_LOC, D), jnp.float32),
            pltpu.SemaphoreType.DMA((2,)),
            pltpu.SemaphoreType.DMA((2,)),
            pltpu.SemaphoreType.DMA((2,)),
            pltpu.SemaphoreType.DMA((4,)),
        ],
        compiler_params=pltpu.CompilerParams(
            collective_id=0,
            vmem_limit_bytes=67_000_000,
        ),
    )(x, router, W1, W2)
